# baseline (speedup 1.0000x reference)
import functools

import jax
import jax.numpy as jnp
from jax import lax
from jax.experimental import pallas as pl
from jax.experimental.pallas import tpu as pltpu
from jax.experimental.pallas import tpu_sc as plsc

_N = 10000
_D = 128
_E = 320000

_NC = 2
_NS = 16
_NW = _NC * _NS
_EPW = _E // _NW
_CH = 64
_NCHUNK = 157
_EPWP = _NCHUNK * _CH
_NP = 10240
_PAD_DST = 10200
_RPT = _NP // _NS
_ZR = 8


def _sc_agg(h, src, dst):
  mesh = plsc.VectorSubcoreMesh(core_axis_name="c", subcore_axis_name="s")

  @functools.partial(
      pl.kernel,
      out_type=(
          jax.ShapeDtypeStruct((_NC, _NP, _D), jnp.float32),
          jax.ShapeDtypeStruct((_NC, _NP, _D), jnp.float32),
      ),
      mesh=mesh,
      scratch_types=[
          pltpu.VMEM((_CH,), jnp.int32),
          pltpu.VMEM((_CH,), jnp.int32),
          pltpu.VMEM((_CH,), jnp.int32),
          pltpu.VMEM((_NCHUNK, _CH), jnp.int32),
          pltpu.VMEM((_CH, _D), jnp.float32),
          pltpu.VMEM((_CH, _D), jnp.float32),
          pltpu.VMEM((_CH, _D), jnp.float32),
          pltpu.VMEM((_ZR, _D), jnp.float32),
          pltpu.VMEM_SHARED((_NP, _D), jnp.float32),
          pltpu.SemaphoreType.DMA,
          pltpu.SemaphoreType.DMA,
          pltpu.SemaphoreType.DMA,
          pltpu.SemaphoreType.DMA,
          pltpu.SemaphoreType.DMA,
          pltpu.SemaphoreType.DMA,
          pltpu.SemaphoreType.DMA,
          pltpu.SemaphoreType.DMA,
          pltpu.SemaphoreType.DMA,
          pltpu.SemaphoreType.DMA,
          pltpu.SemaphoreType.DMA,
      ],
  )
  def k(h_hbm, src_hbm, dst_hbm, agg_out, cnt_out,
        sb0, sb1, sb2, didx_v, r0v, r1v, r2v, zd_v, acc_sp,
        g0, g1, g2, si0, si1, si2, sc0, sc1, sc2, sem_s, sem_z):
    cid = lax.axis_index("c")
    sid = lax.axis_index("s")
    wid = sid * _NC + cid
    sbuf = [sb0, sb1, sb2]
    sisem = [si0, si1, si2]
    rows = [r0v, r1v, r2v]
    gsem = [g0, g1, g2]
    scsem = [sc0, sc1, sc2]

    zero16 = jnp.zeros((16,), jnp.float32)
    one16 = jnp.ones((16,), jnp.float32)

    pltpu.sync_copy(dst_hbm.at[wid], didx_v)

    def fill_zd(t, _):
      zd_v[t // (_D // 16), pl.ds((t % (_D // 16)) * 16, 16)] = zero16
      return 0
    lax.fori_loop(0, _ZR * (_D // 16), fill_zd, 0)

    def zero_own_rows(_unused):
      def zero_slab(z, _):
        r0 = sid * _RPT + z * _ZR
        pltpu.async_copy(zd_v, acc_sp.at[pl.ds(r0, _ZR), :], sem_z)
        return 0
      lax.fori_loop(0, _RPT // _ZR, zero_slab, 0)

      def zero_drain(z, _):
        r0 = sid * _RPT + z * _ZR
        pltpu.make_async_copy(zd_v, acc_sp.at[pl.ds(r0, _ZR), :],
                              sem_z).wait()
        return 0
      lax.fori_loop(0, _RPT // _ZR, zero_drain, 0)

    zero_own_rows(None)
    plsc.subcore_barrier()

    for j in range(3):
      pltpu.async_copy(src_hbm.at[wid, j], sbuf[j], sisem[j])
    for j in range(2):
      pltpu.make_async_copy(src_hbm.at[wid, j], sbuf[j], sisem[j]).wait()
      pltpu.async_copy(h_hbm.at[sbuf[j]], rows[j], gsem[j])

    def triple(q, _):
      for b in range(3):
        i = 3 * q + b
        pltpu.make_async_copy(h_hbm.at[sbuf[b]], rows[b], gsem[b]).wait()

        @pl.when(i + 3 < _NCHUNK)
        def _():
          pltpu.async_copy(src_hbm.at[wid, i + 3], sbuf[b], sisem[b])
        pltpu.async_copy(rows[b], acc_sp.at[didx_v.at[i]], scsem[b],
                         add=True)

        @pl.when(i > 0)
        def _():
          bp = (b + 2) % 3
          pltpu.make_async_copy(rows[bp], acc_sp.at[didx_v.at[i - 1]],
                                scsem[bp]).wait()

        @pl.when(i + 2 < _NCHUNK)
        def _():
          b2 = (b + 2) % 3
          pltpu.make_async_copy(src_hbm.at[wid, i + 2], sbuf[b2],
                                sisem[b2]).wait()
          pltpu.async_copy(h_hbm.at[sbuf[b2]], rows[b2], gsem[b2])
      return 0
    lax.fori_loop(0, (_NCHUNK - 1) // 3, triple, 0)

    pltpu.make_async_copy(h_hbm.at[sbuf[0]], rows[0], gsem[0]).wait()
    pltpu.async_copy(rows[0], acc_sp.at[didx_v.at[_NCHUNK - 1]], scsem[0],
                     add=True)
    pltpu.make_async_copy(rows[2], acc_sp.at[didx_v.at[_NCHUNK - 2]],
                          scsem[2]).wait()
    pltpu.make_async_copy(rows[0], acc_sp.at[didx_v.at[_NCHUNK - 1]],
                          scsem[0]).wait()

    plsc.subcore_barrier()

    r0 = sid * _RPT
    pltpu.sync_copy(acc_sp.at[pl.ds(r0, _RPT), :],
                    agg_out.at[cid, pl.ds(r0, _RPT), :])

    def fill_ones(t, _):
      r0v[t // (_D // 16), pl.ds((t % (_D // 16)) * 16, 16)] = one16
      return 0
    lax.fori_loop(0, _CH * (_D // 16), fill_ones, 0)
    zero_own_rows(None)
    plsc.subcore_barrier()

    def cnt_fire(i, _):
      pltpu.async_copy(r0v, acc_sp.at[didx_v.at[i]], sem_s, add=True)
      return 0
    lax.fori_loop(0, _NCHUNK, cnt_fire, 0)

    def cnt_drain(i, _):
      pltpu.make_async_copy(r0v, acc_sp.at[didx_v.at[i]], sem_s).wait()
      return 0
    lax.fori_loop(0, _NCHUNK, cnt_drain, 0)

    plsc.subcore_barrier()

    pltpu.sync_copy(acc_sp.at[pl.ds(r0, _RPT), :],
                    cnt_out.at[cid, pl.ds(r0, _RPT), :])

  return k(h, src, dst)


def _prep_edges(ei):
  npad = _EPWP - _EPW
  src = ei[0].reshape(_NW, _EPW)
  dst = ei[1].reshape(_NW, _EPW)
  src = jnp.concatenate([src, jnp.zeros((_NW, npad), jnp.int32)], axis=1)
  dst = jnp.concatenate(
      [dst, jnp.full((_NW, npad), _PAD_DST, jnp.int32)], axis=1)
  return (src.reshape(_NW, _NCHUNK, _CH), dst.reshape(_NW, _NCHUNK, _CH))


_BM = 1000


def _gelu(y):
  return 0.5 * y * (1.0 + lax.erf(y * 0.7071067811865476))


def _tc_layer1_body(agg_ref, cnt_ref, h_ref, wl_ref, wr_ref, b_ref, o_ref):
  agg = agg_ref[0] + agg_ref[1]
  cnt = cnt_ref[0, :, 0:1] + cnt_ref[1, :, 0:1]
  mean = agg / jnp.maximum(cnt, 1.0)
  y = (jnp.dot(mean, wl_ref[...], preferred_element_type=jnp.float32)
       + jnp.dot(h_ref[...], wr_ref[...], preferred_element_type=jnp.float32)
       + b_ref[...])
  o_ref[...] = _gelu(y)


def _tc_layer2_body(agg_ref, cnt_ref, h_ref, wl_ref, wr_ref, b_ref,
                    wlin_ref, blin_ref, o_ref):
  agg = agg_ref[0] + agg_ref[1]
  cnt = cnt_ref[0, :, 0:1] + cnt_ref[1, :, 0:1]
  mean = agg / jnp.maximum(cnt, 1.0)
  y = (jnp.dot(mean, wl_ref[...], preferred_element_type=jnp.float32)
       + jnp.dot(h_ref[...], wr_ref[...], preferred_element_type=jnp.float32)
       + b_ref[...])
  g = _gelu(y)
  o_ref[...] = (jnp.dot(g, wlin_ref[...], preferred_element_type=jnp.float32)
                + blin_ref[...])


def _tc_layer(body, agg_parts, cnt_parts, h, mats, out_dim):
  grid = (_N // _BM,)
  in_specs = [
      pl.BlockSpec((_NC, _BM, _D), lambda i: (0, i, 0)),
      pl.BlockSpec((_NC, _BM, _D), lambda i: (0, i, 0)),
      pl.BlockSpec((_BM, _D), lambda i: (i, 0)),
  ]
  args = [agg_parts, cnt_parts, h]
  for m in mats:
    m2 = m if m.ndim == 2 else m.reshape(1, -1)
    in_specs.append(pl.BlockSpec(m2.shape, lambda i: (0, 0)))
    args.append(m2)
  return pl.pallas_call(
      body,
      grid=grid,
      in_specs=in_specs,
      out_specs=pl.BlockSpec((_BM, out_dim), lambda i: (i, 0)),
      out_shape=jax.ShapeDtypeStruct((_N, out_dim), jnp.float32),
  )(*args)


def kernel(x, edge_index_0, edge_index_1, W_l0, W_r0, b0, W_l1, W_r1, b1,
           W_lin, b_lin):
  src0, dst0 = _prep_edges(edge_index_0)
  src1, dst1 = _prep_edges(edge_index_1)
  aggp0, cntp0 = _sc_agg(x, src0, dst0)
  h1 = _tc_layer(_tc_layer1_body, aggp0, cntp0, x, (W_l0, W_r0, b0), _D)
  aggp1, cntp1 = _sc_agg(h1, src1, dst1)
  out = _tc_layer(_tc_layer2_body, aggp1, cntp1, h1,
                  (W_l1, W_r1, b1, W_lin, b_lin), _D)
  return out
